# Initial kernel scaffold; baseline (speedup 1.0000x reference)
import jax
import jax.numpy as jnp
from jax.experimental import pallas as pl


def kernel(x, edge_index, W1l, b1, W1r, W2l, b2, W2r):
    raise NotImplementedError("write your pallas kernel here")


# speedup vs baseline: 6.9027x; 6.9027x over previous
import functools

import jax
import jax.numpy as jnp
from jax import lax
from jax.experimental import pallas as pl
from jax.experimental.pallas import tpu as pltpu
from jax.experimental.pallas import tpu_sc as plsc

N_NODES = 10000
NPAD = 10240
N_EDGES = 320000
NFEAT = 128
CHUNK = 128
NCHUNKS = N_EDGES // CHUNK
NC = 2
NS = 16
NW = NC * NS
ROWS_PER_TILE = NPAD // NS
WB = 128
CNTW = 16
BLK = 2048


def _sc_agg_body(with_cnt, *refs):
    if with_cnt:
        (xh, src, dst, aggs_out, cnt_out,
         acc, idx_s, idx_d, rows, cnt_tile, sem) = refs
    else:
        (xh, src, dst, aggs_out,
         acc, idx_s, idx_d, rows, sem) = refs
        cnt_out = cnt_tile = None

    c = lax.axis_index("c")
    s = lax.axis_index("s")
    wid = c * NS + s
    base = s * ROWS_PER_TILE

    def zero_loop(i, carry):
        for j in range(NFEAT // 16):
            rows[i, pl.ds(j * 16, 16)] = jnp.zeros((16,), jnp.float32)
        return carry
    lax.fori_loop(0, CHUNK, zero_loop, 0)

    if with_cnt:
        def zero_cnt(i, carry):
            cnt_tile[pl.ds(i * 16, 16)] = jnp.zeros((16,), jnp.float32)
            return carry
        lax.fori_loop(0, NPAD // 16, zero_cnt, 0)

    for t in range(ROWS_PER_TILE // WB):
        r0 = base + t * WB
        pltpu.sync_copy(rows.at[pl.ds(0, WB)], acc.at[pl.ds(r0, WB)])

    plsc.subcore_barrier()

    nk = (NCHUNKS - wid + NW - 1) // NW

    def edge_body(k, carry):
        e0 = (wid + k * NW) * CHUNK
        pltpu.sync_copy(src.at[pl.ds(e0, CHUNK)], idx_s)
        pltpu.sync_copy(dst.at[pl.ds(e0, CHUNK)], idx_d)
        pltpu.async_copy(xh.at[idx_s], rows, sem).wait()
        pltpu.sync_copy(rows, acc.at[idx_d], add=True)
        if with_cnt:
            ones16 = jnp.ones((16,), jnp.float32)
            for j in range(CHUNK // 16):
                plsc.addupdate_scatter(
                    cnt_tile, [idx_d[pl.ds(j * 16, 16)]], ones16)
        return carry
    lax.fori_loop(0, nk, edge_body, 0)

    plsc.subcore_barrier()

    out_base = c * NPAD + base
    for t in range(ROWS_PER_TILE // WB):
        r0 = base + t * WB
        pltpu.sync_copy(acc.at[pl.ds(r0, WB)],
                        aggs_out.at[pl.ds(out_base + t * WB, WB)])
    if with_cnt:
        pltpu.sync_copy(cnt_tile, cnt_out.at[wid])


def _make_sc_agg(with_cnt):
    mesh = plsc.VectorSubcoreMesh(core_axis_name="c", subcore_axis_name="s",
                                  num_cores=NC, num_subcores=NS)
    out_type = [jax.ShapeDtypeStruct((NC * NPAD, NFEAT), jnp.float32)]
    if with_cnt:
        out_type.append(jax.ShapeDtypeStruct((NW, NPAD), jnp.float32))
    scratch = [
        pltpu.VMEM_SHARED((NPAD, NFEAT), jnp.float32),
        pltpu.VMEM((CHUNK,), jnp.int32),
        pltpu.VMEM((CHUNK,), jnp.int32),
        pltpu.VMEM((CHUNK, NFEAT), jnp.float32),
    ]
    if with_cnt:
        scratch.append(pltpu.VMEM((NPAD,), jnp.float32))
    scratch.append(pltpu.SemaphoreType.DMA)
    return pl.kernel(functools.partial(_sc_agg_body, with_cnt),
                     out_type=out_type, mesh=mesh, scratch_types=scratch,
                     compiler_params=pltpu.CompilerParams(
                         needs_layout_passes=False))


def _tc_body(relu, a0, a1, cnt, x, wl, b, wr, out):
    agg = a0[...] + a1[...]
    deg = jnp.sum(cnt[...], axis=0)[:, None]
    mean = agg / jnp.maximum(deg, 1.0)
    h = jnp.dot(mean, wl[...], preferred_element_type=jnp.float32,
                precision=lax.Precision.HIGHEST)
    h = h + b[...] + jnp.dot(x[...], wr[...], preferred_element_type=jnp.float32,
                             precision=lax.Precision.HIGHEST)
    if relu:
        h = jnp.maximum(h, 0.0)
    out[...] = h


def _make_tc(relu):
    nblk = NPAD // BLK
    in_specs = [
        pl.BlockSpec((BLK, NFEAT), lambda i: (i, 0)),
        pl.BlockSpec((BLK, NFEAT), lambda i: (i + nblk, 0)),
        pl.BlockSpec((NW, BLK), lambda i: (0, i)),
        pl.BlockSpec((BLK, NFEAT), lambda i: (i, 0)),
        pl.BlockSpec((NFEAT, NFEAT), lambda i: (0, 0)),
        pl.BlockSpec((1, NFEAT), lambda i: (0, 0)),
        pl.BlockSpec((NFEAT, NFEAT), lambda i: (0, 0)),
    ]
    return pl.pallas_call(
        functools.partial(_tc_body, relu),
        grid=(nblk,),
        in_specs=in_specs,
        out_specs=pl.BlockSpec((BLK, NFEAT), lambda i: (i, 0)),
        out_shape=jax.ShapeDtypeStruct((NPAD, NFEAT), jnp.float32),
    )


_sc_agg_cnt = _make_sc_agg(True)
_sc_agg = _make_sc_agg(False)
_tc_layer1 = _make_tc(True)
_tc_layer2 = _make_tc(False)


def kernel(x, edge_index, W1l, b1, W1r, W2l, b2, W2r):
    x = x.astype(jnp.float32)
    ei = edge_index.astype(jnp.int32)
    src, dst = ei[0], ei[1]
    xp = jnp.pad(x, ((0, NPAD - N_NODES), (0, 0)))

    aggs1, cnt2 = _sc_agg_cnt(xp, src, dst)
    h = _tc_layer1(aggs1, aggs1, cnt2, xp,
                   W1l.T, b1.reshape(1, -1), W1r.T)
    (aggs2,) = _sc_agg(h, src, dst)
    out = _tc_layer2(aggs2, aggs2, cnt2, h,
                     W2l.T, b2.reshape(1, -1), W2r.T)
    return out[:N_NODES]
